# baseline (speedup 1.0000x reference)
import functools

import jax
import jax.numpy as jnp
from jax.experimental import pallas as pl
from jax.experimental.pallas import tpu as pltpu

_K_BLK = 4096


def _mm_body(x_ref, w_ref, o_ref, *, k_total, k_blk, nk):
    k = pl.program_id(0)

    @pl.when(k == 0)
    def _init():
        o_ref[...] = jnp.zeros_like(o_ref)

    @pl.when(k < nk - 1)
    def _full():
        o_ref[...] += jax.lax.dot_general(
            x_ref[...], w_ref[...], (((1,), (0,)), ((), ())),
            preferred_element_type=jnp.float32,
        )

    @pl.when(k == nk - 1)
    def _tail():
        rem = k_total - (nk - 1) * k_blk
        xb = x_ref[...]
        wb = w_ref[...]
        col = jax.lax.broadcasted_iota(jnp.int32, xb.shape, 1)
        xb = jnp.where(col < rem, xb, 0.0)
        row = jax.lax.broadcasted_iota(jnp.int32, wb.shape, 0)
        wb = jnp.where(row < rem, wb, 0.0)
        o_ref[...] += jax.lax.dot_general(
            xb, wb, (((1,), (0,)), ((), ())),
            preferred_element_type=jnp.float32,
        )


def kernel(x, W):
    m, k_total = x.shape
    _, n = W.shape
    nk = pl.cdiv(k_total, _K_BLK)
    return pl.pallas_call(
        functools.partial(_mm_body, k_total=k_total, k_blk=_K_BLK, nk=nk),
        grid=(nk,),
        in_specs=[
            pl.BlockSpec((m, _K_BLK), lambda k: (0, k)),
            pl.BlockSpec((_K_BLK, n), lambda k: (k, 0)),
        ],
        out_specs=pl.BlockSpec((m, n), lambda k: (0, 0)),
        out_shape=jax.ShapeDtypeStruct((m, n), jnp.float32),
        compiler_params=pltpu.CompilerParams(
            dimension_semantics=("arbitrary",),
        ),
    )(x, W)


# speedup vs baseline: 1.0064x; 1.0064x over previous
import functools

import jax
import jax.numpy as jnp
from jax.experimental import pallas as pl
from jax.experimental.pallas import tpu as pltpu

_K_BLK = 1024
_NSTREAMS = 4


def _dot(xb, wb):
    return jax.lax.dot_general(
        xb, wb, (((1,), (0,)), ((), ())), preferred_element_type=jnp.float32
    )


def _mm_body(*refs, k_total, k_blk, nk):
    x_refs = refs[:_NSTREAMS]
    w_ref = refs[_NSTREAMS]
    o_ref = refs[_NSTREAMS + 1]
    k = pl.program_id(0)
    span = _NSTREAMS * k_blk

    @pl.when(k == 0)
    def _init():
        o_ref[...] = jnp.zeros_like(o_ref)

    @pl.when(k < nk - 1)
    def _full():
        acc = _dot(x_refs[0][...], w_ref[0:k_blk, :])
        for i in range(1, _NSTREAMS):
            acc += _dot(x_refs[i][...], w_ref[i * k_blk:(i + 1) * k_blk, :])
        o_ref[...] += acc

    @pl.when(k == nk - 1)
    def _tail():
        last_base = (nk - 1) * span
        acc = None
        for i in range(_NSTREAMS):
            rem = k_total - (last_base + i * k_blk)
            if rem <= 0:
                continue
            xb = x_refs[i][...]
            wb = w_ref[i * k_blk:(i + 1) * k_blk, :]
            if rem < k_blk:
                col = jax.lax.broadcasted_iota(jnp.int32, xb.shape, 1)
                xb = jnp.where(col < rem, xb, 0.0)
                row = jax.lax.broadcasted_iota(jnp.int32, wb.shape, 0)
                wb = jnp.where(row < rem, wb, 0.0)
            part = _dot(xb, wb)
            acc = part if acc is None else acc + part
        o_ref[...] += acc


def kernel(x, W):
    m, k_total = x.shape
    _, n = W.shape
    span = _NSTREAMS * _K_BLK
    nk = pl.cdiv(k_total, span)
    nblk_x = pl.cdiv(k_total, _K_BLK)

    def x_spec(i):
        return pl.BlockSpec(
            (m, _K_BLK),
            lambda k, i=i: (0, jnp.minimum(k * _NSTREAMS + i, nblk_x - 1)),
        )

    w_spec = pl.BlockSpec(
        (span, n), lambda k: (jnp.minimum(k, (k_total - 1) // span), 0)
    )
    return pl.pallas_call(
        functools.partial(_mm_body, k_total=k_total, k_blk=_K_BLK, nk=nk),
        grid=(nk,),
        in_specs=[x_spec(i) for i in range(_NSTREAMS)] + [w_spec],
        out_specs=pl.BlockSpec((m, n), lambda k: (0, 0)),
        out_shape=jax.ShapeDtypeStruct((m, n), jnp.float32),
        compiler_params=pltpu.CompilerParams(
            dimension_semantics=("arbitrary",),
        ),
    )(*([x] * _NSTREAMS), W)
